# baseline (speedup 1.0000x reference)
import functools

import jax
import jax.numpy as jnp
from jax import lax
from jax.experimental import pallas as pl
from jax.experimental.pallas import tpu as pltpu
from jax.experimental.pallas import tpu_sc as plsc

VOCAB = 100000
D = 16
B = 1024
CTX = 9
N_ALLIED = 4
GRP = 8
GW = GRP * D

NC, NS = 2, 16
NW = NC * NS
ROWS = B * CTX
R_PER_W = ROWS // NW
CHUNK = 96
NCHUNK = R_PER_W // CHUNK

BV = 2048
NSTRIPE = VOCAB // BV
TAILR = VOCAB - NSTRIPE * BV
NBUF = 4


def _sc_gather(idx_hbm, table_hbm, out_hbm, idx_v, rows_v, sem):
    wid = lax.axis_index("s") * NC + lax.axis_index("c")
    pltpu.sync_copy(idx_hbm.at[wid], idx_v)
    copies = [
        pltpu.async_copy(table_hbm.at[idx_v.at[j]], rows_v.at[j], sem)
        for j in range(NCHUNK)
    ]
    for j, cp in enumerate(copies):
        cp.wait()
        pltpu.sync_copy(
            rows_v.at[j], out_hbm.at[pl.ds(wid * R_PER_W + j * CHUNK, CHUNK)])


_gather_call = functools.partial(
    pl.kernel,
    mesh=plsc.VectorSubcoreMesh(core_axis_name="c", subcore_axis_name="s"),
    out_type=jax.ShapeDtypeStruct((ROWS, GW), jnp.float32),
    scratch_types=[
        pltpu.VMEM((NCHUNK, CHUNK), jnp.int32),
        pltpu.VMEM((NCHUNK, CHUNK, GW), jnp.float32),
        pltpu.SemaphoreType.DMA,
    ],
)(_sc_gather)


def _ctx_kernel(rows_ref, sub_ref, ctx_ref):
    lane = lax.broadcasted_iota(jnp.int32, (B, GW), 1)
    grp_of_lane = lax.shift_right_logical(lane, 4)
    acc_a = jnp.zeros((B, D), jnp.float32)
    acc_e = jnp.zeros((B, D), jnp.float32)
    for t in range(CTX):
        piece = rows_ref[pl.ds(t * B, B), :]
        s = sub_ref[:, t:t + 1]
        m = jnp.where(grp_of_lane == s, piece, 0.0)
        h = m[:, :64] + m[:, 64:]
        q = h[:, :32] + h[:, 32:]
        r = q[:, :D] + q[:, D:]
        norm = jnp.sqrt(jnp.sum(r * r, axis=1, keepdims=True))
        r = r * jnp.minimum(1.0, 1.0 / (norm + 1e-7))
        if t < N_ALLIED:
            acc_a = acc_a + r
        else:
            acc_e = acc_e + r
    ctx_ref[:] = jnp.concatenate(
        [acc_a * (1.0 / N_ALLIED), acc_e * (1.0 / (CTX - N_ALLIED))], axis=1)


def _mm(w_f32, ctx_f32, b_col):
    return lax.dot_general(
        w_f32.astype(jnp.bfloat16), ctx_f32.astype(jnp.bfloat16),
        (((1,), (1,)), ((), ())),
        preferred_element_type=jnp.float32) + b_col


def _head_kernel(ctx_ref, w_ref, b_ref, wt_ref, bt_ref, out_hbm,
                 obuf0, obuf1, obuf2, obuf3, tbuf, sems, tsem):
    v = pl.program_id(0)
    slot = lax.rem(v, NBUF)
    ctx = ctx_ref[:]
    obufs = [obuf0, obuf1, obuf2, obuf3]

    for s in range(NBUF):
        @pl.when(jnp.logical_and(v >= NBUF, slot == s))
        def _():
            pltpu.make_async_copy(
                obufs[s], out_hbm.at[pl.ds(0, BV), :], sems.at[s]).wait()

        @pl.when(slot == s)
        def _():
            obufs[s][:] = _mm(w_ref[:], ctx, b_ref[:])
            pltpu.make_async_copy(
                obufs[s], out_hbm.at[pl.ds(v * BV, BV), :],
                sems.at[s]).start()

    @pl.when(v == NSTRIPE - 1)
    def _():
        tbuf[:] = _mm(wt_ref[:], ctx, bt_ref[:])
        pltpu.make_async_copy(
            tbuf, out_hbm.at[pl.ds(NSTRIPE * BV, TAILR), :], tsem).start()
        pltpu.make_async_copy(
            tbuf, out_hbm.at[pl.ds(NSTRIPE * BV, TAILR), :], tsem).wait()
        for s in range(NBUF):
            pltpu.make_async_copy(
                obufs[s], out_hbm.at[pl.ds(0, BV), :], sems.at[s]).wait()


def kernel(ctx_heroes, t_table, head_w, head_b):
    idx = ctx_heroes.astype(jnp.int32)
    grp_idx = (idx // GRP).T.reshape(NW, NCHUNK, CHUNK)
    sub = idx % GRP
    rows = _gather_call(grp_idx, t_table.reshape(VOCAB // GRP, GW))

    ctx = pl.pallas_call(
        _ctx_kernel,
        out_shape=jax.ShapeDtypeStruct((B, 2 * D), jnp.float32),
    )(rows, sub)

    b_col = head_b.reshape(VOCAB, 1)
    logits_t = pl.pallas_call(
        _head_kernel,
        grid=(NSTRIPE,),
        in_specs=[
            pl.BlockSpec((B, 2 * D), lambda v: (0, 0)),
            pl.BlockSpec((BV, 2 * D), lambda v: (v, 0)),
            pl.BlockSpec((BV, 1), lambda v: (v, 0)),
            pl.BlockSpec((TAILR, 2 * D), lambda v: (0, 0)),
            pl.BlockSpec((TAILR, 1), lambda v: (0, 0)),
        ],
        out_specs=pl.BlockSpec(memory_space=pl.ANY),
        out_shape=jax.ShapeDtypeStruct((VOCAB, B), jnp.float32),
        scratch_shapes=[
            pltpu.VMEM((BV, B), jnp.float32),
            pltpu.VMEM((BV, B), jnp.float32),
            pltpu.VMEM((BV, B), jnp.float32),
            pltpu.VMEM((BV, B), jnp.float32),
            pltpu.VMEM((TAILR, B), jnp.float32),
            pltpu.SemaphoreType.DMA((NBUF,)),
            pltpu.SemaphoreType.DMA,
        ],
    )(ctx, head_w, b_col,
      head_w[NSTRIPE * BV:], b_col[NSTRIPE * BV:])
    return logits_t.T


# speedup vs baseline: 1.0286x; 1.0286x over previous
import functools

import jax
import jax.numpy as jnp
from jax import lax
from jax.experimental import pallas as pl
from jax.experimental.pallas import tpu as pltpu
from jax.experimental.pallas import tpu_sc as plsc

VOCAB = 100000
D = 16
B = 1024
CTX = 9
N_ALLIED = 4
GRP = 8
GW = GRP * D

NC, NS = 2, 16
NW = NC * NS
ROWS = B * CTX
R_PER_W = ROWS // NW
CHUNK = 96
NCHUNK = R_PER_W // CHUNK

BV = 4096
NV = (VOCAB + BV - 1) // BV


def _sc_gather(idx_hbm, table_hbm, out_hbm, idx_v, rows_v, sem):
    wid = lax.axis_index("s") * NC + lax.axis_index("c")
    pltpu.sync_copy(idx_hbm.at[wid], idx_v)
    copies = [
        pltpu.async_copy(table_hbm.at[idx_v.at[j]], rows_v.at[j], sem)
        for j in range(NCHUNK)
    ]
    for j, cp in enumerate(copies):
        cp.wait()
        pltpu.sync_copy(
            rows_v.at[j], out_hbm.at[pl.ds(wid * R_PER_W + j * CHUNK, CHUNK)])


_gather_call = functools.partial(
    pl.kernel,
    mesh=plsc.VectorSubcoreMesh(core_axis_name="c", subcore_axis_name="s"),
    out_type=jax.ShapeDtypeStruct((ROWS, GW), jnp.float32),
    scratch_types=[
        pltpu.VMEM((NCHUNK, CHUNK), jnp.int32),
        pltpu.VMEM((NCHUNK, CHUNK, GW), jnp.float32),
        pltpu.SemaphoreType.DMA,
    ],
)(_sc_gather)


def _ctx_kernel(rows_ref, sub_ref, ctx_ref):
    lane = lax.broadcasted_iota(jnp.int32, (B, GW), 1)
    grp_of_lane = lax.shift_right_logical(lane, 4)
    acc_a = jnp.zeros((B, D), jnp.float32)
    acc_e = jnp.zeros((B, D), jnp.float32)
    for t in range(CTX):
        piece = rows_ref[pl.ds(t * B, B), :]
        s = sub_ref[:, t:t + 1]
        m = jnp.where(grp_of_lane == s, piece, 0.0)
        h = m[:, :64] + m[:, 64:]
        q = h[:, :32] + h[:, 32:]
        r = q[:, :D] + q[:, D:]
        norm = jnp.sqrt(jnp.sum(r * r, axis=1, keepdims=True))
        r = r * jnp.minimum(1.0, 1.0 / (norm + 1e-7))
        if t < N_ALLIED:
            acc_a = acc_a + r
        else:
            acc_e = acc_e + r
    ctx_ref[:] = jnp.concatenate(
        [acc_a * (1.0 / N_ALLIED), acc_e * (1.0 / (CTX - N_ALLIED))], axis=1)


def _head_kernel(ctx_ref, w_ref, b_ref, out_ref):
    out_ref[:] = lax.dot_general(
        w_ref[:].astype(jnp.bfloat16), ctx_ref[:].astype(jnp.bfloat16),
        (((1,), (1,)), ((), ())),
        preferred_element_type=jnp.float32) + b_ref[:]


def kernel(ctx_heroes, t_table, head_w, head_b):
    idx = ctx_heroes.astype(jnp.int32)
    grp_idx = (idx // GRP).T.reshape(NW, NCHUNK, CHUNK)
    sub = idx % GRP
    rows = _gather_call(grp_idx, t_table.reshape(VOCAB // GRP, GW))

    ctx = pl.pallas_call(
        _ctx_kernel,
        out_shape=jax.ShapeDtypeStruct((B, 2 * D), jnp.float32),
    )(rows, sub)

    logits_t = pl.pallas_call(
        _head_kernel,
        grid=(NV,),
        in_specs=[
            pl.BlockSpec((B, 2 * D), lambda v: (0, 0)),
            pl.BlockSpec((BV, 2 * D), lambda v: (v, 0)),
            pl.BlockSpec((BV, 1), lambda v: (v, 0)),
        ],
        out_specs=pl.BlockSpec((BV, B), lambda v: (v, 0)),
        out_shape=jax.ShapeDtypeStruct((VOCAB, B), jnp.float32),
    )(ctx, head_w, head_b.reshape(VOCAB, 1))
    return logits_t.T
